# Initial kernel scaffold; baseline (speedup 1.0000x reference)
import functools

import jax
import jax.numpy as jnp
from jax import lax
from jax.experimental import pallas as pl
from jax.experimental.pallas import tpu as pltpu
from jax.experimental.pallas import tpu_sc as plsc

_NC = 2
_NS = 16
_NW = _NC * _NS


def _make_sc_gather(M, D, B):
  bpw = B // _NW
  mesh = plsc.VectorSubcoreMesh(core_axis_name="c", subcore_axis_name="s")

  @functools.partial(
      pl.kernel,
      out_type=(
          jax.ShapeDtypeStruct((B, D), jnp.float32),
          jax.ShapeDtypeStruct((B,), jnp.int32),
          jax.ShapeDtypeStruct((B,), jnp.int32),
      ),
      mesh=mesh,
      scratch_types=[
          pltpu.VMEM((bpw,), jnp.int32),
          pltpu.VMEM((bpw, D), jnp.float32),
          pltpu.VMEM((bpw,), jnp.int32),
          pltpu.VMEM((bpw,), jnp.int32),
          pltpu.SemaphoreType.DMA,
          pltpu.SemaphoreType.DMA,
          pltpu.SemaphoreType.DMA,
      ],
  )
  def gather_kernel(mem_hbm, idx_hbm, cnt_hbm, lm_hbm,
                    sel_out, cnt_out, lm_out,
                    idx_v, rows_v, cnt_v, lm_v, sem0, sem1, sem2):
    wid = lax.axis_index("s") * _NC + lax.axis_index("c")
    base = wid * bpw
    pltpu.sync_copy(idx_hbm.at[pl.ds(base, bpw)], idx_v)
    c0 = pltpu.async_copy(mem_hbm.at[idx_v], rows_v, sem0)
    c1 = pltpu.async_copy(cnt_hbm.at[idx_v], cnt_v, sem1)
    c2 = pltpu.async_copy(lm_hbm.at[idx_v], lm_v, sem2)
    c0.wait()
    c1.wait()
    c2.wait()
    pltpu.sync_copy(rows_v, sel_out.at[pl.ds(base, bpw)])
    pltpu.sync_copy(cnt_v, cnt_out.at[pl.ds(base, bpw)])
    pltpu.sync_copy(lm_v, lm_out.at[pl.ds(base, bpw)])

  return gather_kernel


def _make_sc_scatter(D, B):
  bpw = B // _NW
  mesh = plsc.VectorSubcoreMesh(core_axis_name="c", subcore_axis_name="s")

  @functools.partial(
      pl.kernel,
      out_type=(),
      mesh=mesh,
      scratch_types=[
          pltpu.VMEM((bpw,), jnp.int32),
          pltpu.VMEM((bpw, D), jnp.float32),
          pltpu.SemaphoreType.DMA,
      ],
  )
  def scatter_kernel(mem_ref, vals_hbm, idx_hbm, idx_v, rows_v, sem):
    wid = lax.axis_index("s") * _NC + lax.axis_index("c")
    base = wid * bpw
    pltpu.sync_copy(idx_hbm.at[pl.ds(base, bpw)], idx_v)
    pltpu.sync_copy(vals_hbm.at[pl.ds(base, bpw)], rows_v)
    pltpu.async_copy(rows_v, mem_ref.at[idx_v], sem).wait()

  return scatter_kernel


def _bucket(v):
  lg = (5
        + (v >= 8).astype(jnp.int32)
        + (v >= 16).astype(jnp.int32)
        + (v >= 32).astype(jnp.int32)
        + (v >= 64).astype(jnp.int32))
  return jnp.where(v <= 4, jnp.clip(v, 0, 9), lg)


def _dense_body(mi_ref, b2_ref, qr_ref, wq_ref, bq_ref, msel_ref, cnt_ref,
                lm_ref, act_ref, mscore_ref, w1m_ref, w1q_ref, w1p_ref,
                w1d_ref, w1c_ref, w1a_ref, b1_ref, w2_ref, dt_ref, ct_ref,
                at_ref, delta_ref):
  q = jnp.dot(qr_ref[...], wq_ref[...],
              preferred_element_type=jnp.float32) + bq_ref[...]
  msel = msel_ref[...]
  h = jnp.dot(msel, w1m_ref[...], preferred_element_type=jnp.float32)
  h += jnp.dot(q, w1q_ref[...], preferred_element_type=jnp.float32)
  h += jnp.dot(msel * q, w1p_ref[...], preferred_element_type=jnp.float32)

  db = _bucket(mi_ref[0, 0] - lm_ref[...])
  cb = _bucket(cnt_ref[...])
  iota10 = lax.broadcasted_iota(jnp.int32, (1, 10), 1)
  iota5 = lax.broadcasted_iota(jnp.int32, (1, 5), 1)
  ohd = (db == iota10).astype(jnp.float32)
  ohc = (cb == iota10).astype(jnp.float32)
  oha = (act_ref[...] == iota5).astype(jnp.float32)
  h += jnp.dot(ohd, jnp.dot(dt_ref[...], w1d_ref[...],
                            preferred_element_type=jnp.float32),
               preferred_element_type=jnp.float32)
  h += jnp.dot(ohc, jnp.dot(ct_ref[...], w1c_ref[...],
                            preferred_element_type=jnp.float32),
               preferred_element_type=jnp.float32)
  h += jnp.dot(oha, jnp.dot(at_ref[...], w1a_ref[...],
                            preferred_element_type=jnp.float32),
               preferred_element_type=jnp.float32)
  h = jnp.maximum(h + b1_ref[...], 0.0)

  score = (jnp.sum(h * w2_ref[...], axis=1, keepdims=True)
           + b2_ref[0, 0] + mscore_ref[...])
  gate = 1.0 / (1.0 + jnp.exp(-score))
  cnt_f = cnt_ref[...].astype(jnp.float32)
  delta_ref[...] = (gate / (cnt_f + 1.0)) * (q - msel)


def _tc_dense(R, mi, b2, query_raw, W_q, b_q, mem_sel, cnt2, lm2, act2,
              mscore2, w1m, w1q, w1p, w1d, w1c, w1a, b1r, w2r, dt, ct, at):
  B, QIN = query_raw.shape
  D = W_q.shape[1]
  H = b1r.shape[1]
  nb = B // R
  full = lambda shape: pl.BlockSpec(shape, lambda i: (0,) * len(shape))
  row = lambda shape: pl.BlockSpec(shape, lambda i: (i,) + (0,) * (len(shape) - 1))
  smem = pl.BlockSpec(memory_space=pltpu.SMEM)
  return pl.pallas_call(
      _dense_body,
      grid=(nb,),
      in_specs=[
          smem, smem,
          row((R, QIN)), full((QIN, D)), full((1, D)),
          row((R, D)), row((R, 1)), row((R, 1)), row((R, 1)), row((R, 1)),
          full((D, H)), full((D, H)), full((D, H)),
          full((20, H)), full((20, H)), full((20, H)),
          full((1, H)), full((1, H)),
          full((10, 20)), full((10, 20)), full((5, 20)),
      ],
      out_specs=row((R, D)),
      out_shape=jax.ShapeDtypeStruct((B, D), jnp.float32),
  )(mi, b2, query_raw, W_q, b_q, mem_sel, cnt2, lm2, act2, mscore2,
    w1m, w1q, w1p, w1d, w1c, w1a, b1r, w2r, dt, ct, at)


def _combine_body(ci_ref, call_ref, dfull_ref, msel_ref, out_ref):
  eq = (ci_ref[...] == call_ref[...]).astype(jnp.float32)
  out_ref[...] = msel_ref[...] + jnp.dot(
      eq, dfull_ref[...], preferred_element_type=jnp.float32)


def _tc_combine(R, cell2, cell_row, delta, mem_sel):
  B, D = delta.shape
  nb = B // R
  return pl.pallas_call(
      _combine_body,
      grid=(nb,),
      in_specs=[
          pl.BlockSpec((R, 1), lambda i: (i, 0)),
          pl.BlockSpec((1, B), lambda i: (0, 0)),
          pl.BlockSpec((B, D), lambda i: (0, 0)),
          pl.BlockSpec((R, D), lambda i: (i, 0)),
      ],
      out_specs=pl.BlockSpec((R, D), lambda i: (i, 0)),
      out_shape=jax.ShapeDtypeStruct((B, D), jnp.float32),
  )(cell2, cell_row, delta, mem_sel)


def kernel(mem_vectors, ent_counter, last_mention_idx, query_raw, ment_score,
           cell_idx, last_action, ment_idx, W_q, b_q, W1, b1, W2, b2,
           distance_table, counter_table, action_table):
  M, D = mem_vectors.shape
  B = query_raw.shape[0]
  H = b1.shape[0]
  R = 512

  cell = cell_idx.astype(jnp.int32)
  cnt_i = ent_counter.astype(jnp.int32)
  lm_i = last_mention_idx.astype(jnp.int32)

  mem_sel, cnt_sel, lm_sel = _make_sc_gather(M, D, B)(mem_vectors, cell,
                                                      cnt_i, lm_i)

  mi = jnp.asarray(ment_idx, jnp.int32).reshape(1, 1)
  b2r = b2.astype(jnp.float32).reshape(1, 1)
  delta = _tc_dense(
      R, mi, b2r, query_raw, W_q, b_q.reshape(1, D), mem_sel,
      cnt_sel.reshape(B, 1), lm_sel.reshape(B, 1),
      last_action.astype(jnp.int32).reshape(B, 1), ment_score.reshape(B, 1),
      W1[0:D], W1[D:2 * D], W1[2 * D:3 * D],
      W1[3 * D:3 * D + 20], W1[3 * D + 20:3 * D + 40], W1[3 * D + 40:],
      b1.reshape(1, H), W2.reshape(1, H),
      distance_table, counter_table, action_table)

  vals = _tc_combine(R, cell.reshape(B, 1), cell.reshape(1, B), delta,
                     mem_sel)

  mem_ref = jax.new_ref(mem_vectors)
  _make_sc_scatter(D, B)(mem_ref, vals, cell)
  return mem_ref[...]


# speedup vs baseline: 4.8394x; 4.8394x over previous
import functools

import jax
import jax.numpy as jnp
from jax import lax
from jax.experimental import pallas as pl
from jax.experimental.pallas import tpu as pltpu
from jax.experimental.pallas import tpu_sc as plsc

_NC = 2
_NS = 16
_NW = _NC * _NS


def _make_sc_gather_ints(B):
  bpw = B // _NW
  mesh = plsc.VectorSubcoreMesh(core_axis_name="c", subcore_axis_name="s")

  @functools.partial(
      pl.kernel,
      out_type=(
          jax.ShapeDtypeStruct((B,), jnp.int32),
          jax.ShapeDtypeStruct((B,), jnp.int32),
      ),
      mesh=mesh,
      scratch_types=[
          pltpu.VMEM((bpw,), jnp.int32),
          pltpu.VMEM((bpw,), jnp.int32),
          pltpu.VMEM((bpw,), jnp.int32),
          pltpu.SemaphoreType.DMA,
          pltpu.SemaphoreType.DMA,
      ],
      compiler_params=pltpu.CompilerParams(use_tc_tiling_on_sc=False),
  )
  def gather_ints_kernel(idx_hbm, cnt_hbm, lm_hbm, cnt_out, lm_out,
                         idx_v, cnt_v, lm_v, sem1, sem2):
    wid = lax.axis_index("s") * _NC + lax.axis_index("c")
    base = wid * bpw
    pltpu.sync_copy(idx_hbm.at[pl.ds(base, bpw)], idx_v)
    c1 = pltpu.async_copy(cnt_hbm.at[idx_v], cnt_v, sem1)
    c2 = pltpu.async_copy(lm_hbm.at[idx_v], lm_v, sem2)
    c1.wait()
    c2.wait()
    pltpu.sync_copy(cnt_v, cnt_out.at[pl.ds(base, bpw)])
    pltpu.sync_copy(lm_v, lm_out.at[pl.ds(base, bpw)])

  return gather_ints_kernel


def _make_sc_gather_rows(M, D, B):
  bpw = B // _NW
  mesh = plsc.VectorSubcoreMesh(core_axis_name="c", subcore_axis_name="s")

  @functools.partial(
      pl.kernel,
      out_type=jax.ShapeDtypeStruct((B, D), jnp.float32),
      mesh=mesh,
      scratch_types=[
          pltpu.VMEM((bpw,), jnp.int32),
          pltpu.VMEM((bpw, D), jnp.float32),
          pltpu.SemaphoreType.DMA,
      ],
      compiler_params=pltpu.CompilerParams(needs_layout_passes=False),
  )
  def gather_rows_kernel(mem_hbm, idx_hbm, sel_out,
                         idx_v, rows_v, sem0):
    wid = lax.axis_index("s") * _NC + lax.axis_index("c")
    base = wid * bpw
    pltpu.sync_copy(idx_hbm.at[pl.ds(base, bpw)], idx_v)
    lane = lax.broadcasted_iota(jnp.int32, (16,), 0)

    @pl.loop(0, bpw // 16)
    def _issue(c):
      off = pl.multiple_of(c * 16, 16)
      chunk = idx_v[pl.ds(off, 16)]
      for l in range(16):
        s = jnp.sum(jnp.where(lane == l, chunk, 0))
        k = c * 16 + l
        pltpu.async_copy(mem_hbm.at[pl.ds(s, 1)], rows_v.at[pl.ds(k, 1)],
                         sem0)

    @pl.loop(0, bpw)
    def _drain(k):
      pltpu.make_async_copy(mem_hbm.at[pl.ds(0, 1)],
                            rows_v.at[pl.ds(k, 1)], sem0).wait()

    pltpu.sync_copy(rows_v, sel_out.at[pl.ds(base, bpw)])

  return gather_rows_kernel


def _make_sc_scatter(D, B):
  bpw = B // _NW
  mesh = plsc.VectorSubcoreMesh(core_axis_name="c", subcore_axis_name="s")

  @functools.partial(
      pl.kernel,
      out_type=(),
      mesh=mesh,
      scratch_types=[
          pltpu.VMEM((bpw,), jnp.int32),
          pltpu.VMEM((bpw, D), jnp.float32),
          pltpu.SemaphoreType.DMA,
      ],
      compiler_params=pltpu.CompilerParams(needs_layout_passes=False),
  )
  def scatter_kernel(mem_ref, vals_hbm, idx_hbm, idx_v, rows_v, sem):
    wid = lax.axis_index("s") * _NC + lax.axis_index("c")
    base = wid * bpw
    pltpu.sync_copy(idx_hbm.at[pl.ds(base, bpw)], idx_v)
    pltpu.sync_copy(vals_hbm.at[pl.ds(base, bpw)], rows_v)
    lane = lax.broadcasted_iota(jnp.int32, (16,), 0)

    @pl.loop(0, bpw // 16)
    def _issue(c):
      off = pl.multiple_of(c * 16, 16)
      chunk = idx_v[pl.ds(off, 16)]
      for l in range(16):
        s = jnp.sum(jnp.where(lane == l, chunk, 0))
        k = c * 16 + l
        pltpu.async_copy(rows_v.at[pl.ds(k, 1)], mem_ref.at[pl.ds(s, 1)],
                         sem)

    @pl.loop(0, bpw)
    def _drain(k):
      pltpu.make_async_copy(rows_v.at[pl.ds(k, 1)],
                            mem_ref.at[pl.ds(0, 1)], sem).wait()

  return scatter_kernel


def _bucket(v):
  lg = (5
        + (v >= 8).astype(jnp.int32)
        + (v >= 16).astype(jnp.int32)
        + (v >= 32).astype(jnp.int32)
        + (v >= 64).astype(jnp.int32))
  return jnp.where(v <= 4, jnp.clip(v, 0, 9), lg)


def _dense_body(mi_ref, b2_ref, qr_ref, wq_ref, bq_ref, msel_ref, cnt_ref,
                lm_ref, act_ref, mscore_ref, w1m_ref, w1q_ref, w1p_ref,
                w1d_ref, w1c_ref, w1a_ref, b1_ref, w2_ref, dt_ref, ct_ref,
                at_ref, delta_ref):
  q = jnp.dot(qr_ref[...], wq_ref[...],
              preferred_element_type=jnp.float32) + bq_ref[...]
  msel = msel_ref[...]
  h = jnp.dot(msel, w1m_ref[...], preferred_element_type=jnp.float32)
  h += jnp.dot(q, w1q_ref[...], preferred_element_type=jnp.float32)
  h += jnp.dot(msel * q, w1p_ref[...], preferred_element_type=jnp.float32)

  db = _bucket(mi_ref[0, 0] - lm_ref[...])
  cb = _bucket(cnt_ref[...])
  iota10 = lax.broadcasted_iota(jnp.int32, (1, 10), 1)
  iota5 = lax.broadcasted_iota(jnp.int32, (1, 5), 1)
  ohd = (db == iota10).astype(jnp.float32)
  ohc = (cb == iota10).astype(jnp.float32)
  oha = (act_ref[...] == iota5).astype(jnp.float32)
  h += jnp.dot(ohd, jnp.dot(dt_ref[...], w1d_ref[...],
                            preferred_element_type=jnp.float32),
               preferred_element_type=jnp.float32)
  h += jnp.dot(ohc, jnp.dot(ct_ref[...], w1c_ref[...],
                            preferred_element_type=jnp.float32),
               preferred_element_type=jnp.float32)
  h += jnp.dot(oha, jnp.dot(at_ref[...], w1a_ref[...],
                            preferred_element_type=jnp.float32),
               preferred_element_type=jnp.float32)
  h = jnp.maximum(h + b1_ref[...], 0.0)

  score = (jnp.sum(h * w2_ref[...], axis=1, keepdims=True)
           + b2_ref[0, 0] + mscore_ref[...])
  gate = 1.0 / (1.0 + jnp.exp(-score))
  cnt_f = cnt_ref[...].astype(jnp.float32)
  delta_ref[...] = (gate / (cnt_f + 1.0)) * (q - msel)


def _tc_dense(R, mi, b2, query_raw, W_q, b_q, mem_sel, cnt2, lm2, act2,
              mscore2, w1m, w1q, w1p, w1d, w1c, w1a, b1r, w2r, dt, ct, at):
  B, QIN = query_raw.shape
  D = W_q.shape[1]
  H = b1r.shape[1]
  nb = B // R
  full = lambda shape: pl.BlockSpec(shape, lambda i: (0,) * len(shape))
  row = lambda shape: pl.BlockSpec(shape, lambda i: (i,) + (0,) * (len(shape) - 1))
  smem = pl.BlockSpec(memory_space=pltpu.SMEM)
  return pl.pallas_call(
      _dense_body,
      grid=(nb,),
      in_specs=[
          smem, smem,
          row((R, QIN)), full((QIN, D)), full((1, D)),
          row((R, D)), row((R, 1)), row((R, 1)), row((R, 1)), row((R, 1)),
          full((D, H)), full((D, H)), full((D, H)),
          full((20, H)), full((20, H)), full((20, H)),
          full((1, H)), full((1, H)),
          full((10, 20)), full((10, 20)), full((5, 20)),
      ],
      out_specs=row((R, D)),
      out_shape=jax.ShapeDtypeStruct((B, D), jnp.float32),
  )(mi, b2, query_raw, W_q, b_q, mem_sel, cnt2, lm2, act2, mscore2,
    w1m, w1q, w1p, w1d, w1c, w1a, b1r, w2r, dt, ct, at)


def _combine_body(ci_ref, call_ref, dfull_ref, msel_ref, out_ref):
  eq = (ci_ref[...] == call_ref[...]).astype(jnp.float32)
  out_ref[...] = msel_ref[...] + jnp.dot(
      eq, dfull_ref[...], preferred_element_type=jnp.float32)


def _tc_combine(R, cell2, cell_row, delta, mem_sel):
  B, D = delta.shape
  nb = B // R
  return pl.pallas_call(
      _combine_body,
      grid=(nb,),
      in_specs=[
          pl.BlockSpec((R, 1), lambda i: (i, 0)),
          pl.BlockSpec((1, B), lambda i: (0, 0)),
          pl.BlockSpec((B, D), lambda i: (0, 0)),
          pl.BlockSpec((R, D), lambda i: (i, 0)),
      ],
      out_specs=pl.BlockSpec((R, D), lambda i: (i, 0)),
      out_shape=jax.ShapeDtypeStruct((B, D), jnp.float32),
  )(cell2, cell_row, delta, mem_sel)


def kernel(mem_vectors, ent_counter, last_mention_idx, query_raw, ment_score,
           cell_idx, last_action, ment_idx, W_q, b_q, W1, b1, W2, b2,
           distance_table, counter_table, action_table):
  M, D = mem_vectors.shape
  B = query_raw.shape[0]
  H = b1.shape[0]
  R = 512

  cell = cell_idx.astype(jnp.int32)
  cnt_i = ent_counter.astype(jnp.int32)
  lm_i = last_mention_idx.astype(jnp.int32)

  mem_sel = _make_sc_gather_rows(M, D, B)(mem_vectors, cell)
  cnt_sel, lm_sel = _make_sc_gather_ints(B)(cell, cnt_i, lm_i)

  mi = jnp.asarray(ment_idx, jnp.int32).reshape(1, 1)
  b2r = b2.astype(jnp.float32).reshape(1, 1)
  delta = _tc_dense(
      R, mi, b2r, query_raw, W_q, b_q.reshape(1, D), mem_sel,
      cnt_sel.reshape(B, 1), lm_sel.reshape(B, 1),
      last_action.astype(jnp.int32).reshape(B, 1), ment_score.reshape(B, 1),
      W1[0:D], W1[D:2 * D], W1[2 * D:3 * D],
      W1[3 * D:3 * D + 20], W1[3 * D + 20:3 * D + 40], W1[3 * D + 40:],
      b1.reshape(1, H), W2.reshape(1, H),
      distance_table, counter_table, action_table)

  vals = _tc_combine(R, cell.reshape(B, 1), cell.reshape(1, B), delta,
                     mem_sel)

  mem_ref = jax.new_ref(mem_vectors)
  _make_sc_scatter(D, B)(mem_ref, vals, cell)
  return mem_ref[...]


# speedup vs baseline: 5.7913x; 1.1967x over previous
import functools

import jax
import jax.numpy as jnp
from jax import lax
from jax.experimental import pallas as pl
from jax.experimental.pallas import tpu as pltpu
from jax.experimental.pallas import tpu_sc as plsc

_NC = 2
_NS = 16
_NW = _NC * _NS


def _make_sc_gather_ints(B):
  bpw = B // _NW
  mesh = plsc.VectorSubcoreMesh(core_axis_name="c", subcore_axis_name="s")

  @functools.partial(
      pl.kernel,
      out_type=(
          jax.ShapeDtypeStruct((B,), jnp.int32),
          jax.ShapeDtypeStruct((B,), jnp.int32),
      ),
      mesh=mesh,
      scratch_types=[
          pltpu.VMEM((bpw,), jnp.int32),
          pltpu.VMEM((bpw,), jnp.int32),
          pltpu.VMEM((bpw,), jnp.int32),
          pltpu.SemaphoreType.DMA,
          pltpu.SemaphoreType.DMA,
      ],
      compiler_params=pltpu.CompilerParams(use_tc_tiling_on_sc=False),
  )
  def gather_ints_kernel(idx_hbm, cnt_hbm, lm_hbm, cnt_out, lm_out,
                         idx_v, cnt_v, lm_v, sem1, sem2):
    wid = lax.axis_index("s") * _NC + lax.axis_index("c")
    base = wid * bpw
    pltpu.sync_copy(idx_hbm.at[pl.ds(base, bpw)], idx_v)
    c1 = pltpu.async_copy(cnt_hbm.at[idx_v], cnt_v, sem1)
    c2 = pltpu.async_copy(lm_hbm.at[idx_v], lm_v, sem2)
    c1.wait()
    c2.wait()
    pltpu.sync_copy(cnt_v, cnt_out.at[pl.ds(base, bpw)])
    pltpu.sync_copy(lm_v, lm_out.at[pl.ds(base, bpw)])

  return gather_ints_kernel


def _make_sc_gather_rows(M, D, B):
  bpw = B // _NW
  mesh = plsc.VectorSubcoreMesh(core_axis_name="c", subcore_axis_name="s")

  @functools.partial(
      pl.kernel,
      out_type=jax.ShapeDtypeStruct((B, D), jnp.float32),
      mesh=mesh,
      scratch_types=[
          pltpu.VMEM((bpw,), jnp.int32),
          pltpu.VMEM((bpw, D), jnp.float32),
          pltpu.SemaphoreType.DMA,
      ],
      compiler_params=pltpu.CompilerParams(needs_layout_passes=False),
  )
  def gather_rows_kernel(mem_hbm, idx_hbm, sel_out,
                         idx_v, rows_v, sem0):
    wid = lax.axis_index("s") * _NC + lax.axis_index("c")
    base = wid * bpw
    pltpu.sync_copy(idx_hbm.at[pl.ds(base, bpw)], idx_v)
    lane = lax.broadcasted_iota(jnp.int32, (16,), 0)

    @pl.loop(0, bpw // 16)
    def _issue(c):
      off = pl.multiple_of(c * 16, 16)
      chunk = idx_v[pl.ds(off, 16)]
      for l in range(16):
        s = jnp.sum(jnp.where(lane == l, chunk, 0))
        k = c * 16 + l
        pltpu.async_copy(mem_hbm.at[pl.ds(s, 1)], rows_v.at[pl.ds(k, 1)],
                         sem0)

    @pl.loop(0, bpw)
    def _drain(k):
      pltpu.make_async_copy(mem_hbm.at[pl.ds(0, 1)],
                            rows_v.at[pl.ds(k, 1)], sem0).wait()

    pltpu.sync_copy(rows_v, sel_out.at[pl.ds(base, bpw)])

  return gather_rows_kernel


def _make_sc_scatter(D, B):
  bpw = B // _NW
  mesh = plsc.VectorSubcoreMesh(core_axis_name="c", subcore_axis_name="s")

  @functools.partial(
      pl.kernel,
      out_type=(),
      mesh=mesh,
      scratch_types=[
          pltpu.VMEM((bpw,), jnp.int32),
          pltpu.VMEM((bpw, D), jnp.float32),
          pltpu.SemaphoreType.DMA,
      ],
      compiler_params=pltpu.CompilerParams(needs_layout_passes=False),
  )
  def scatter_kernel(mem_ref, vals_hbm, idx_hbm, idx_v, rows_v, sem):
    wid = lax.axis_index("s") * _NC + lax.axis_index("c")
    base = wid * bpw
    pltpu.sync_copy(idx_hbm.at[pl.ds(base, bpw)], idx_v)
    pltpu.sync_copy(vals_hbm.at[pl.ds(base, bpw)], rows_v)
    lane = lax.broadcasted_iota(jnp.int32, (16,), 0)

    @pl.loop(0, bpw // 16)
    def _issue(c):
      off = pl.multiple_of(c * 16, 16)
      chunk = idx_v[pl.ds(off, 16)]
      for l in range(16):
        s = jnp.sum(jnp.where(lane == l, chunk, 0))
        k = c * 16 + l
        pltpu.async_copy(rows_v.at[pl.ds(k, 1)], mem_ref.at[pl.ds(s, 1)],
                         sem)

    @pl.loop(0, bpw)
    def _drain(k):
      pltpu.make_async_copy(rows_v.at[pl.ds(k, 1)],
                            mem_ref.at[pl.ds(0, 1)], sem).wait()

  return scatter_kernel


def _bucket(v):
  lg = (5
        + (v >= 8).astype(jnp.int32)
        + (v >= 16).astype(jnp.int32)
        + (v >= 32).astype(jnp.int32)
        + (v >= 64).astype(jnp.int32))
  return jnp.where(v <= 4, jnp.clip(v, 0, 9), lg)


def _dense_body(mi_ref, b2_ref, qr_ref, wq_ref, bq_ref, msel_ref, cnt_ref,
                lm_ref, act_ref, mscore_ref, w1m_ref, w1q_ref, w1p_ref,
                w1d_ref, w1c_ref, w1a_ref, b1_ref, w2_ref, dt_ref, ct_ref,
                at_ref, delta_ref):
  q = jnp.dot(qr_ref[...], wq_ref[...],
              preferred_element_type=jnp.float32) + bq_ref[...]
  msel = msel_ref[...]
  h = jnp.dot(msel, w1m_ref[...], preferred_element_type=jnp.float32)
  h += jnp.dot(q, w1q_ref[...], preferred_element_type=jnp.float32)
  h += jnp.dot(msel * q, w1p_ref[...], preferred_element_type=jnp.float32)

  db = _bucket(mi_ref[0, 0] - lm_ref[...])
  cb = _bucket(cnt_ref[...])
  iota10 = lax.broadcasted_iota(jnp.int32, (1, 10), 1)
  iota5 = lax.broadcasted_iota(jnp.int32, (1, 5), 1)
  ohd = (db == iota10).astype(jnp.float32)
  ohc = (cb == iota10).astype(jnp.float32)
  oha = (act_ref[...] == iota5).astype(jnp.float32)
  h += jnp.dot(ohd, jnp.dot(dt_ref[...], w1d_ref[...],
                            preferred_element_type=jnp.float32),
               preferred_element_type=jnp.float32)
  h += jnp.dot(ohc, jnp.dot(ct_ref[...], w1c_ref[...],
                            preferred_element_type=jnp.float32),
               preferred_element_type=jnp.float32)
  h += jnp.dot(oha, jnp.dot(at_ref[...], w1a_ref[...],
                            preferred_element_type=jnp.float32),
               preferred_element_type=jnp.float32)
  h = jnp.maximum(h + b1_ref[...], 0.0)

  score = (jnp.sum(h * w2_ref[...], axis=1, keepdims=True)
           + b2_ref[0, 0] + mscore_ref[...])
  gate = 1.0 / (1.0 + jnp.exp(-score))
  cnt_f = cnt_ref[...].astype(jnp.float32)
  delta_ref[...] = (gate / (cnt_f + 1.0)) * (q - msel)


def _tc_dense(R, mi, b2, query_raw, W_q, b_q, mem_sel, cnt2, lm2, act2,
              mscore2, w1m, w1q, w1p, w1d, w1c, w1a, b1r, w2r, dt, ct, at):
  B, QIN = query_raw.shape
  D = W_q.shape[1]
  H = b1r.shape[1]
  nb = B // R
  full = lambda shape: pl.BlockSpec(shape, lambda i: (0,) * len(shape))
  row = lambda shape: pl.BlockSpec(shape, lambda i: (i,) + (0,) * (len(shape) - 1))
  smem = pl.BlockSpec(memory_space=pltpu.SMEM)
  return pl.pallas_call(
      _dense_body,
      grid=(nb,),
      in_specs=[
          smem, smem,
          row((R, QIN)), full((QIN, D)), full((1, D)),
          row((R, D)), row((R, 1)), row((R, 1)), row((R, 1)), row((R, 1)),
          full((D, H)), full((D, H)), full((D, H)),
          full((20, H)), full((20, H)), full((20, H)),
          full((1, H)), full((1, H)),
          full((10, 20)), full((10, 20)), full((5, 20)),
      ],
      out_specs=row((R, D)),
      out_shape=jax.ShapeDtypeStruct((B, D), jnp.float32),
  )(mi, b2, query_raw, W_q, b_q, mem_sel, cnt2, lm2, act2, mscore2,
    w1m, w1q, w1p, w1d, w1c, w1a, b1r, w2r, dt, ct, at)


def _combine_body(ci_ref, call_ref, dfull_ref, msel_ref, out_ref):
  eq = (ci_ref[...] == call_ref[...]).astype(jnp.float32)
  out_ref[...] = msel_ref[...] + jnp.dot(
      eq, dfull_ref[...], preferred_element_type=jnp.float32)


def _tc_combine(R, cell2, cell_row, delta, mem_sel):
  B, D = delta.shape
  nb = B // R
  return pl.pallas_call(
      _combine_body,
      grid=(nb,),
      in_specs=[
          pl.BlockSpec((R, 1), lambda i: (i, 0)),
          pl.BlockSpec((1, B), lambda i: (0, 0)),
          pl.BlockSpec((B, D), lambda i: (0, 0)),
          pl.BlockSpec((R, D), lambda i: (i, 0)),
      ],
      out_specs=pl.BlockSpec((R, D), lambda i: (i, 0)),
      out_shape=jax.ShapeDtypeStruct((B, D), jnp.float32),
  )(cell2, cell_row, delta, mem_sel)


def _tr_body(x_ref, y_ref):
  y_ref[...] = jnp.swapaxes(x_ref[...], 0, 1)


def _tc_transpose(x, BL=2048):
  P, Q = x.shape
  if Q >= P:
    return pl.pallas_call(
        _tr_body,
        grid=(pl.cdiv(Q, BL),),
        in_specs=[pl.BlockSpec((P, BL), lambda j: (0, j))],
        out_specs=pl.BlockSpec((BL, P), lambda j: (j, 0)),
        out_shape=jax.ShapeDtypeStruct((Q, P), jnp.float32),
    )(x)
  return pl.pallas_call(
      _tr_body,
      grid=(pl.cdiv(P, BL),),
      in_specs=[pl.BlockSpec((BL, Q), lambda j: (j, 0))],
      out_specs=pl.BlockSpec((Q, BL), lambda j: (0, j)),
      out_shape=jax.ShapeDtypeStruct((Q, P), jnp.float32),
  )(x)


def kernel(mem_vectors, ent_counter, last_mention_idx, query_raw, ment_score,
           cell_idx, last_action, ment_idx, W_q, b_q, W1, b1, W2, b2,
           distance_table, counter_table, action_table):
  M, D = mem_vectors.shape
  B = query_raw.shape[0]
  H = b1.shape[0]
  R = 512

  cell = cell_idx.astype(jnp.int32)
  cnt_i = ent_counter.astype(jnp.int32)
  lm_i = last_mention_idx.astype(jnp.int32)

  mem_t = jnp.swapaxes(mem_vectors, 0, 1)
  mem_row = _tc_transpose(mem_t)

  mem_sel = _make_sc_gather_rows(M, D, B)(mem_row, cell)
  cnt_sel, lm_sel = _make_sc_gather_ints(B)(cell, cnt_i, lm_i)

  mi = jnp.asarray(ment_idx, jnp.int32).reshape(1, 1)
  b2r = b2.astype(jnp.float32).reshape(1, 1)
  delta = _tc_dense(
      R, mi, b2r, query_raw, W_q, b_q.reshape(1, D), mem_sel,
      cnt_sel.reshape(B, 1), lm_sel.reshape(B, 1),
      last_action.astype(jnp.int32).reshape(B, 1), ment_score.reshape(B, 1),
      W1[0:D], W1[D:2 * D], W1[2 * D:3 * D],
      W1[3 * D:3 * D + 20], W1[3 * D + 20:3 * D + 40], W1[3 * D + 40:],
      b1.reshape(1, H), W2.reshape(1, H),
      distance_table, counter_table, action_table)

  vals = _tc_combine(R, cell.reshape(B, 1), cell.reshape(1, B), delta,
                     mem_sel)

  mem_ref = jax.new_ref(mem_row)
  _make_sc_scatter(D, B)(mem_ref, vals, cell)
  out_t = _tc_transpose(mem_ref[...])
  return jnp.swapaxes(out_t, 0, 1)
